# baseline (speedup 1.0000x reference)
import jax
import jax.numpy as jnp
from jax import lax
from jax.experimental import pallas as pl

_SEQ = 128
_D = 64
_C = 64
_BS = 128
_BQ = 128


def _pool_project(x, w):
    pooled = jnp.sum(x, axis=2) * (1.0 / _SEQ)
    return jnp.dot(pooled, w, preferred_element_type=jnp.float32)


def _seg_partial(lbl, emb, n):
    onehot = (lbl[:, None] == lax.broadcasted_iota(jnp.int32, (n, _C), 1)
              ).astype(jnp.float32)
    part_sums = lax.dot_general(onehot, emb, (((0,), (0,)), ((), ())),
                                preferred_element_type=jnp.float32)
    ones_col = jnp.ones((n, 1), jnp.float32)
    part_counts = lax.dot_general(onehot, ones_col, (((0,), (0,)), ((), ())),
                                  preferred_element_type=jnp.float32)
    return part_sums, part_counts


def _support_body(labels_a_ref, labels_b_ref, xa_ref, xb_ref, w_ref,
                  sums_ref, counts_ref):
    i = pl.program_id(0)
    emb_a = _pool_project(xa_ref[...], w_ref[...])
    emb_b = _pool_project(xb_ref[...], w_ref[...])
    sums_a, counts_a = _seg_partial(labels_a_ref[0, 0, :], emb_a, _BS)
    sums_b, counts_b = _seg_partial(labels_b_ref[0, 0, :], emb_b, _BS)
    part_sums = sums_a + sums_b
    part_counts = counts_a + counts_b

    @pl.when(i == 0)
    def _():
        sums_ref[...] = part_sums
        counts_ref[...] = part_counts

    @pl.when(i > 0)
    def _():
        sums_ref[...] += part_sums
        counts_ref[...] += part_counts


def _query_body(xa_ref, xb_ref, w_ref, b_ref, sums_ref, counts_ref,
                logits_ta_ref, logits_tb_ref, protos_ref):
    j = pl.program_id(0)
    counts = counts_ref[...]
    denom = jnp.maximum(counts, 1.0)
    protos = (sums_ref[...] + counts * b_ref[...]) / denom

    @pl.when(j == 0)
    def _():
        protos_ref[...] = protos

    p2 = jnp.sum(protos * protos, axis=1, keepdims=True)
    ones_row = jnp.ones((1, _D), jnp.float32)
    for x_ref, out_ref in ((xa_ref, logits_ta_ref), (xb_ref, logits_tb_ref)):
        qe = _pool_project(x_ref[...], w_ref[...]) + b_ref[...]
        q2t = lax.dot_general(ones_row, qe * qe, (((1,), (1,)), ((), ())),
                              preferred_element_type=jnp.float32)
        cross_t = lax.dot_general(protos, qe, (((1,), (1,)), ((), ())),
                                  preferred_element_type=jnp.float32)
        out_ref[...] = -(p2 + q2t - 2.0 * cross_t + 1e-8)


@jax.jit
def kernel(support, support_labels, query, W, b):
    n_sup = support.shape[0]
    n_q = query.shape[0]
    hs = n_sup // 2
    hq = n_q // 2
    nbs = hs // _BS
    nbq = hq // _BQ
    sup_t = support.transpose(0, 2, 1)
    q_t = query.transpose(0, 2, 1)
    labels = support_labels.astype(jnp.int32).reshape(2 * nbs, 1, _BS)
    b_row = b.reshape(1, _D)

    xspec = pl.BlockSpec((_BS, _D, _SEQ), lambda i: (i, 0, 0))
    xspec_hi = pl.BlockSpec((_BS, _D, _SEQ), lambda i, n=nbs: (n + i, 0, 0))
    sums, counts = pl.pallas_call(
        _support_body,
        grid=(nbs,),
        in_specs=[
            pl.BlockSpec((1, 1, _BS), lambda i: (i, 0, 0)),
            pl.BlockSpec((1, 1, _BS), lambda i, n=nbs: (n + i, 0, 0)),
            xspec,
            xspec_hi,
            pl.BlockSpec((_D, _D), lambda i: (0, 0)),
        ],
        out_specs=[
            pl.BlockSpec((_C, _D), lambda i: (0, 0)),
            pl.BlockSpec((_C, 1), lambda i: (0, 0)),
        ],
        out_shape=[
            jax.ShapeDtypeStruct((_C, _D), jnp.float32),
            jax.ShapeDtypeStruct((_C, 1), jnp.float32),
        ],
    )(labels, labels, sup_t, sup_t, W)

    qspec = pl.BlockSpec((_BQ, _D, _SEQ), lambda j: (j, 0, 0))
    qspec_hi = pl.BlockSpec((_BQ, _D, _SEQ), lambda j, n=nbq: (n + j, 0, 0))
    logits_ta, logits_tb, protos = pl.pallas_call(
        _query_body,
        grid=(nbq,),
        in_specs=[
            qspec,
            qspec_hi,
            pl.BlockSpec((_D, _D), lambda j: (0, 0)),
            pl.BlockSpec((1, _D), lambda j: (0, 0)),
            pl.BlockSpec((_C, _D), lambda j: (0, 0)),
            pl.BlockSpec((_C, 1), lambda j: (0, 0)),
        ],
        out_specs=[
            pl.BlockSpec((_C, _BQ), lambda j: (0, j)),
            pl.BlockSpec((_C, _BQ), lambda j: (0, j)),
            pl.BlockSpec((_C, _D), lambda j: (0, 0)),
        ],
        out_shape=[
            jax.ShapeDtypeStruct((_C, hq), jnp.float32),
            jax.ShapeDtypeStruct((_C, hq), jnp.float32),
            jax.ShapeDtypeStruct((_C, _D), jnp.float32),
        ],
    )(q_t, q_t, W, b_row, sums, counts)

    logits = jnp.concatenate([logits_ta.T, logits_tb.T], axis=0)
    return (logits, protos)
